# baseline (speedup 1.0000x reference)
import functools

import numpy as np
import jax
import jax.numpy as jnp
from jax.experimental import pallas as pl
from jax.experimental.pallas import tpu as pltpu
from jax.experimental.pallas import tpu_sc as plsc

D = 300
PD = 384
NEG = 0.01
_SCALE = float(1.0 / np.sqrt(float(D)))

_NC, _NS = 2, 16
_NW = _NC * _NS
_GCHUNK = 128

_BF = jnp.bfloat16


def _lrelu(x):
    return jnp.where(x >= 0, x, NEG * x)


def _bf(x):
    return x.astype(_BF)


def _mm(a, b):
    return jax.lax.dot_general(_bf(a), _bf(b), (((1,), (0,)), ((), ())),
                               preferred_element_type=jnp.float32)


def _mm_t(a, b):
    return jax.lax.dot_general(_bf(a), _bf(b), (((1,), (1,)), ((), ())),
                               preferred_element_type=jnp.float32)


def _pre_body(x1_ref, x2_ref, ea1_ref, ea2_ref, ei1_ref, ei2_ref,
              w_ref, b_ref, g_ref, e1_ref, e2_ref, idx_ref):
    n = x1_ref.shape[0]
    for gi, x_ref in enumerate((x1_ref, x2_ref)):
        x = x_ref[...]
        for j in range(4):
            base = (gi * 4 + j) * n
            wj = 5 * gi + j
            g_ref[base:base + n, :] = _mm(x, w_ref[wj]) + b_ref[wj:wj + 1, :]
    for gi, (ea_ref, e_ref) in enumerate(((ea1_ref, e1_ref),
                                          (ea2_ref, e2_ref))):
        wj = 5 * gi + 4
        e_ref[...] = _bf(_mm(ea_ref[...], w_ref[wj]) + b_ref[wj:wj + 1, :])
    idx_ref[0:1, :] = ei1_ref[1:2, :]
    idx_ref[1:2, :] = ei2_ref[1:2, :] + 4 * n


def _sc_gather(table, idx2d, g, ne):
    nch = ne // (_NW * _GCHUNK)
    mesh = plsc.VectorSubcoreMesh(core_axis_name="c", subcore_axis_name="s")

    @functools.partial(
        pl.kernel, mesh=mesh,
        out_type=jax.ShapeDtypeStruct((ne, PD), jnp.float32),
        scratch_types=[pltpu.VMEM((_GCHUNK,), jnp.int32),
                       pltpu.VMEM((_GCHUNK, PD), jnp.float32),
                       pltpu.SemaphoreType.DMA],
    )
    def knl(table_hbm, idx_hbm, out_hbm, idx_v, rows_v, sem):
        wid = jax.lax.axis_index("s") * _NC + jax.lax.axis_index("c")

        @pl.loop(0, nch)
        def _(j):
            base = (wid * nch + j) * _GCHUNK
            pltpu.sync_copy(idx_hbm.at[g, pl.ds(base, _GCHUNK)], idx_v)
            pltpu.async_copy(table_hbm.at[idx_v], rows_v, sem).wait()
            pltpu.sync_copy(rows_v, out_hbm.at[pl.ds(base, _GCHUNK)])

    return knl(table, idx2d)


def _fin_body(qd_ref, q_ref, k_ref, v_ref, s_ref, e_ref, ei_ref, o_ref):
    n = o_ref.shape[0]
    ne = e_ref.shape[0]
    e = e_ref[...]
    qd = qd_ref[...]

    src = ei_ref[0:1, :]
    dst = ei_ref[1:2, :]
    row_ids = jax.lax.broadcasted_iota(jnp.int32, (n, ne), 0)
    msrc = (row_ids == src).astype(_BF)
    mdst_b = (row_ids == dst)

    sqk = _mm_t(q_ref[...], k_ref[...])
    rows = _mm(sqk, msrc)
    alpha1 = jnp.sum(jnp.where(mdst_b, rows, 0.0), axis=0, keepdims=True)
    alpha2 = _mm_t(jnp.ones((1, PD), jnp.float32),
                   qd * e.astype(jnp.float32))
    alpha = (alpha1 + alpha2) * _SCALE

    a_dense = jnp.where(mdst_b, alpha, -jnp.inf)
    amax = jnp.max(a_dense, axis=1, keepdims=True)
    amax = jnp.where(amax == -jnp.inf, 0.0, amax)
    p = jnp.exp(a_dense - amax)
    denom = jnp.sum(p, axis=1, keepdims=True)
    pn = p / (denom + 1e-16)

    c = _mm_t(pn, msrc)
    out = _mm(c, v_ref[...]) + _mm(pn, e) + s_ref[...]
    o_ref[...] = _lrelu(out)


def _cross_body(x1_ref, x2_ref, w_ref, b_ref,
                w1_ref, b1_ref, w2_ref, b2_ref, w3_ref, b3_ref, o_ref):
    x1 = x1_ref[...]
    x2 = x2_ref[...]

    def conv(xd, xs, o):
        ec = (jnp.sum(w_ref[o + 4].astype(jnp.float32), axis=0, keepdims=True)
              + b_ref[o + 4:o + 5, :])
        qd = _mm(xd, w_ref[o + 0]) + b_ref[o + 0:o + 1, :]
        ks = _mm(xs, w_ref[o + 1]) + b_ref[o + 1:o + 2, :] + ec
        vs = _mm(xs, w_ref[o + 2]) + b_ref[o + 2:o + 3, :] + ec
        sd = _mm(xd, w_ref[o + 3]) + b_ref[o + 3:o + 4, :]
        al = _mm_t(qd, ks) * _SCALE
        amax = jnp.max(al, axis=1, keepdims=True)
        p = jnp.exp(al - amax)
        denom = jnp.sum(p, axis=1, keepdims=True)
        pn = p / (denom + 1e-16)
        return _lrelu(_mm(pn, vs) + sd)

    x1n = conv(x1, x2, 0)
    x2n = conv(x2, x1, 5)

    p1 = jnp.mean(x1n, axis=0, keepdims=True)
    p2 = jnp.mean(x2n, axis=0, keepdims=True)
    xc = jnp.concatenate([p1, p2], axis=1)
    h = _lrelu(_mm(xc, w1_ref[...]) + b1_ref[...])
    h = _lrelu(_mm(h, w2_ref[...]) + b2_ref[...])
    o = _mm(h, w3_ref[...]) + b3_ref[...]
    o = 1.0 / (1.0 + jnp.exp(-o))

    o_ref[...] = jnp.zeros((8, 2 * PD), jnp.float32)
    o_ref[0:1, 0:PD] = p1
    o_ref[1:2, 0:PD] = p2
    o_ref[2:3, 0:128] = o


def _pack(pa, pb):
    names = ('q', 'k', 'v', 's', 'e')
    w = jnp.stack([
        jnp.pad(p['W' + nm][0], ((0, PD - D), (0, PD - D)))
        for p in (pa, pb) for nm in names]).astype(_BF)
    b = jnp.pad(
        jnp.stack([jnp.pad(p['b' + nm][0], (0, PD - D))
                   for p in (pa, pb) for nm in names]),
        ((0, 6), (0, 0)))
    return w, b


def kernel(x_1, x_2, edge_idx_1, edge_idx_2, edge_attr_1, edge_attr_2, params):
    n = x_1.shape[0]
    ne = edge_idx_1.shape[1]
    f32 = jnp.float32

    padf = lambda a: jnp.pad(a, ((0, 0), (0, PD - D)))
    x1p_in = padf(x_1)
    x2p_in = padf(x_2)
    ea1 = padf(edge_attr_1)
    ea2 = padf(edge_attr_2)
    ei1 = jnp.pad(edge_idx_1.astype(jnp.int32), ((0, 6), (0, 0)))
    ei2 = jnp.pad(edge_idx_2.astype(jnp.int32), ((0, 6), (0, 0)))
    wI, bI = _pack(params['TSA'], params['GSA'])
    wC, bC = _pack(params['TCA'], params['GCA'])

    gtab, e1, e2, idx2d = pl.pallas_call(
        _pre_body,
        out_shape=[jax.ShapeDtypeStruct((8 * n, PD), f32),
                   jax.ShapeDtypeStruct((ne, PD), _BF),
                   jax.ShapeDtypeStruct((ne, PD), _BF),
                   jax.ShapeDtypeStruct((8, ne), jnp.int32)])(
        x1p_in, x2p_in, ea1, ea2, ei1, ei2, wI, bI)

    qd1 = _sc_gather(gtab, idx2d, 0, ne)
    qd2 = _sc_gather(gtab, idx2d, 1, ne)

    def fin(gi, qd, e, ei):
        blk = lambda r: pl.BlockSpec((n, PD), lambda i, r=r: (r, 0))
        return pl.pallas_call(
            _fin_body,
            grid=(1,),
            in_specs=[
                pl.BlockSpec((ne, PD), lambda i: (0, 0)),
                blk(4 * gi + 0), blk(4 * gi + 1),
                blk(4 * gi + 2), blk(4 * gi + 3),
                pl.BlockSpec((ne, PD), lambda i: (0, 0)),
                pl.BlockSpec((8, ne), lambda i: (0, 0)),
            ],
            out_specs=pl.BlockSpec((n, PD), lambda i: (0, 0)),
            out_shape=jax.ShapeDtypeStruct((n, PD), f32),
        )(qd, gtab, gtab, gtab, gtab, e, ei)

    x1p = fin(0, qd1, e1, ei1)
    x2p = fin(1, qd2, e2, ei2)

    m = params['mlp']
    w1p = jnp.concatenate(
        [jnp.pad(m['W1'][:D], ((0, PD - D), (0, 40))),
         jnp.pad(m['W1'][D:], ((0, PD - D), (0, 40)))], axis=0)
    b1p = jnp.pad(m['b1'].reshape(1, -1), ((0, 0), (0, 40)))
    w2p = jnp.pad(m['W2'], ((0, 40), (0, PD - D)))
    b2p = jnp.pad(m['b2'].reshape(1, -1), ((0, 0), (0, PD - D)))
    w3p = jnp.pad(m['W3'], ((0, PD - D), (0, 127)))
    b3p = jnp.pad(m['b3'].reshape(1, 1), ((0, 0), (0, 127)))

    packed = pl.pallas_call(
        _cross_body, out_shape=jax.ShapeDtypeStruct((8, 2 * PD), f32))(
        x1p, x2p, wC, bC, w1p, b1p, w2p, b2p, w3p, b3p)

    p1 = packed[0, :D]
    p2 = packed[1, :D]
    out = packed[2, :1]
    return (p1, p2, out)


# speedup vs baseline: 1.3080x; 1.3080x over previous
import functools

import numpy as np
import jax
import jax.numpy as jnp
from jax.experimental import pallas as pl
from jax.experimental.pallas import tpu as pltpu
from jax.experimental.pallas import tpu_sc as plsc

D = 300
PD = 384
NEG = 0.01
_SCALE = float(1.0 / np.sqrt(float(D)))

_NC, _NS = 2, 16
_NW = _NC * _NS
_GCHUNK = 128

_BF = jnp.bfloat16


def _lrelu(x):
    return jnp.where(x >= 0, x, NEG * x)


def _bf(x):
    return x.astype(_BF)


def _mm(a, b):
    return jax.lax.dot_general(_bf(a), _bf(b), (((1,), (0,)), ((), ())),
                               preferred_element_type=jnp.float32)


def _mm_t(a, b):
    return jax.lax.dot_general(_bf(a), _bf(b), (((1,), (1,)), ((), ())),
                               preferred_element_type=jnp.float32)


def _preq_body(x1_ref, x2_ref, ei1_ref, ei2_ref, wq1_ref, wq2_ref, b_ref,
               gq_ref, idx_ref):
    n = x1_ref.shape[0]
    gq_ref[...] = jnp.zeros(gq_ref.shape, jnp.float32)
    gq_ref[0:n, 0:D] = _mm(x1_ref[...], wq1_ref[0]) + b_ref[0:1, :]
    gq_ref[n:2 * n, 0:D] = _mm(x2_ref[...], wq2_ref[0]) + b_ref[5:6, :]
    idx_ref[0:1, :] = ei1_ref[1:2, :]
    idx_ref[1:2, :] = ei2_ref[1:2, :] + n


def _prekvse_body(x1_ref, x2_ref, ea1_ref, ea2_ref, w_ref, b_ref,
                  gkvs_ref, e1_ref, e2_ref):
    n = x1_ref.shape[0]
    gkvs_ref[...] = jnp.zeros(gkvs_ref.shape, jnp.float32)
    for gi, x_ref in enumerate((x1_ref, x2_ref)):
        x = x_ref[...]
        for j in range(3):
            base = (gi * 3 + j) * n
            wj = 5 * gi + 1 + j
            gkvs_ref[base:base + n, 0:D] = (
                _mm(x, w_ref[wj]) + b_ref[wj:wj + 1, :])
    for gi, (ea_ref, e_ref) in enumerate(((ea1_ref, e1_ref),
                                          (ea2_ref, e2_ref))):
        wj = 5 * gi + 4
        e_ref[...] = jnp.zeros(e_ref.shape, _BF)
        e_ref[:, 0:D] = _bf(_mm(ea_ref[...], w_ref[wj]) + b_ref[wj:wj + 1, :])


def _sc_gather(table, idx2d, g, ne):
    nch = ne // (_NW * _GCHUNK)
    mesh = plsc.VectorSubcoreMesh(core_axis_name="c", subcore_axis_name="s")

    @functools.partial(
        pl.kernel, mesh=mesh,
        out_type=jax.ShapeDtypeStruct((ne, PD), jnp.float32),
        scratch_types=[pltpu.VMEM((_GCHUNK,), jnp.int32),
                       pltpu.VMEM((_GCHUNK, PD), jnp.float32),
                       pltpu.SemaphoreType.DMA],
    )
    def knl(table_hbm, idx_hbm, out_hbm, idx_v, rows_v, sem):
        wid = jax.lax.axis_index("s") * _NC + jax.lax.axis_index("c")

        @pl.loop(0, nch)
        def _(j):
            base = (wid * nch + j) * _GCHUNK
            pltpu.sync_copy(idx_hbm.at[g, pl.ds(base, _GCHUNK)], idx_v)
            pltpu.async_copy(table_hbm.at[idx_v], rows_v, sem).wait()
            pltpu.sync_copy(rows_v, out_hbm.at[pl.ds(base, _GCHUNK)])

    return knl(table, idx2d)


def _fin_body(qd_ref, q_ref, k_ref, v_ref, s_ref, e_ref, ei_ref, o_ref):
    n = o_ref.shape[0]
    ne = e_ref.shape[0]
    e = e_ref[...]
    qd = qd_ref[...]

    src = ei_ref[0:1, :]
    dst = ei_ref[1:2, :]
    row_ids = jax.lax.broadcasted_iota(jnp.int32, (n, ne), 0)
    msrc = (row_ids == src).astype(_BF)
    mdst_b = (row_ids == dst)

    sqk = _mm_t(q_ref[...], k_ref[...])
    rows = _mm(sqk, msrc)
    alpha1 = jnp.sum(jnp.where(mdst_b, rows, 0.0), axis=0, keepdims=True)
    alpha2 = _mm_t(jnp.ones((1, PD), jnp.float32),
                   qd * e.astype(jnp.float32))
    alpha = (alpha1 + alpha2) * _SCALE

    a_dense = jnp.where(mdst_b, alpha, -jnp.inf)
    amax = jnp.max(a_dense, axis=1, keepdims=True)
    amax = jnp.where(amax == -jnp.inf, 0.0, amax)
    p = jnp.exp(a_dense - amax)
    denom = jnp.sum(p, axis=1, keepdims=True)
    pn = p / (denom + 1e-16)

    c = _mm_t(pn, msrc)
    out = _mm(c, v_ref[...]) + _mm(pn, e) + s_ref[...]
    o_ref[...] = _lrelu(out)[:, 0:D]


def _cross_body(x1_ref, x2_ref, w_ref, b_ref,
                w1_ref, b1_ref, w2_ref, b2_ref, w3_ref, b3_ref, o_ref):
    x1 = x1_ref[...]
    x2 = x2_ref[...]

    def conv(xd, xs, o):
        ec = (jnp.sum(w_ref[o + 4].astype(jnp.float32), axis=0, keepdims=True)
              + b_ref[o + 4:o + 5, :])
        qd = _mm(xd, w_ref[o + 0]) + b_ref[o + 0:o + 1, :]
        ks = _mm(xs, w_ref[o + 1]) + b_ref[o + 1:o + 2, :] + ec
        vs = _mm(xs, w_ref[o + 2]) + b_ref[o + 2:o + 3, :] + ec
        sd = _mm(xd, w_ref[o + 3]) + b_ref[o + 3:o + 4, :]
        al = _mm_t(qd, ks) * _SCALE
        amax = jnp.max(al, axis=1, keepdims=True)
        p = jnp.exp(al - amax)
        denom = jnp.sum(p, axis=1, keepdims=True)
        pn = p / (denom + 1e-16)
        return _lrelu(_mm(pn, vs) + sd)

    x1n = conv(x1, x2, 0)
    x2n = conv(x2, x1, 5)

    p1 = jnp.mean(x1n, axis=0, keepdims=True)
    p2 = jnp.mean(x2n, axis=0, keepdims=True)
    xc = jnp.concatenate([p1, p2], axis=1)
    h = _lrelu(_mm(xc, w1_ref[...]) + b1_ref[...])
    h = _lrelu(_mm(h, w2_ref[...]) + b2_ref[...])
    o = _mm(h, w3_ref[...]) + b3_ref[...]
    o = 1.0 / (1.0 + jnp.exp(-o))

    o_ref[...] = jnp.zeros((8, 2 * D), jnp.float32)
    o_ref[0:1, 0:D] = p1
    o_ref[1:2, 0:D] = p2
    o_ref[2:3, 0:128] = jnp.broadcast_to(o, (1, 128))


def _pack(pa, pb):
    names = ('q', 'k', 'v', 's', 'e')
    w = jnp.stack([p['W' + nm][0] for p in (pa, pb) for nm in names])
    b = jnp.stack([p['b' + nm][0] for p in (pa, pb) for nm in names])
    return w.astype(_BF), b


def kernel(x_1, x_2, edge_idx_1, edge_idx_2, edge_attr_1, edge_attr_2, params):
    n = x_1.shape[0]
    ne = edge_idx_1.shape[1]
    f32 = jnp.float32

    x1b = _bf(x_1)
    x2b = _bf(x_2)
    ea1b = _bf(edge_attr_1)
    ea2b = _bf(edge_attr_2)
    ei1 = edge_idx_1.astype(jnp.int32)
    ei2 = edge_idx_2.astype(jnp.int32)
    wI, bI = _pack(params['TSA'], params['GSA'])
    wC, bC = _pack(params['TCA'], params['GCA'])

    gq, idx2d = pl.pallas_call(
        _preq_body,
        grid=(1,),
        in_specs=[
            pl.BlockSpec((n, D), lambda i: (0, 0)),
            pl.BlockSpec((n, D), lambda i: (0, 0)),
            pl.BlockSpec((2, ne), lambda i: (0, 0)),
            pl.BlockSpec((2, ne), lambda i: (0, 0)),
            pl.BlockSpec((1, D, D), lambda i: (0, 0, 0)),
            pl.BlockSpec((1, D, D), lambda i: (5, 0, 0)),
            pl.BlockSpec((10, D), lambda i: (0, 0)),
        ],
        out_specs=[pl.BlockSpec((2 * n, PD), lambda i: (0, 0)),
                   pl.BlockSpec((2, ne), lambda i: (0, 0))],
        out_shape=[jax.ShapeDtypeStruct((2 * n, PD), f32),
                   jax.ShapeDtypeStruct((2, ne), jnp.int32)])(
        x1b, x2b, ei1, ei2, wI, wI, bI)

    qd1 = _sc_gather(gq, idx2d, 0, ne)
    qd2 = _sc_gather(gq, idx2d, 1, ne)

    gkvs, e1, e2 = pl.pallas_call(
        _prekvse_body,
        out_shape=[jax.ShapeDtypeStruct((6 * n, PD), f32),
                   jax.ShapeDtypeStruct((ne, PD), _BF),
                   jax.ShapeDtypeStruct((ne, PD), _BF)])(
        x1b, x2b, ea1b, ea2b, wI, bI)

    def fin(gi, qd, e, ei):
        return pl.pallas_call(
            _fin_body,
            grid=(1,),
            in_specs=[
                pl.BlockSpec((ne, PD), lambda i: (0, 0)),
                pl.BlockSpec((n, PD), lambda i, gi=gi: (gi, 0)),
                pl.BlockSpec((n, PD), lambda i, gi=gi: (3 * gi, 0)),
                pl.BlockSpec((n, PD), lambda i, gi=gi: (3 * gi + 1, 0)),
                pl.BlockSpec((n, PD), lambda i, gi=gi: (3 * gi + 2, 0)),
                pl.BlockSpec((ne, PD), lambda i: (0, 0)),
                pl.BlockSpec((2, ne), lambda i: (0, 0)),
            ],
            out_specs=pl.BlockSpec((n, D), lambda i: (0, 0)),
            out_shape=jax.ShapeDtypeStruct((n, D), f32),
        )(qd, gq, gkvs, gkvs, gkvs, e, ei)

    x1p = fin(0, qd1, e1, ei1)
    x2p = fin(1, qd2, e2, ei2)

    m = params['mlp']
    packed = pl.pallas_call(
        _cross_body, out_shape=jax.ShapeDtypeStruct((8, 2 * D), f32))(
        x1p, x2p, wC, bC,
        m['W1'], m['b1'].reshape(1, -1), m['W2'], m['b2'].reshape(1, -1),
        m['W3'], m['b3'].reshape(1, 1))

    p1 = packed[0, :D]
    p2 = packed[1, :D]
    out = packed[2, :1]
    return (p1, p2, out)
